# Initial kernel scaffold; baseline (speedup 1.0000x reference)
import jax
import jax.numpy as jnp
from jax.experimental import pallas as pl


def kernel(x, edge_index, Wl0, bl0, Wr0, br0, att0, bo0, rW0, rb0, g0, be0, Wl1, bl1, Wr1, br1, att1, bo1, rW1, rb1, g1, be1, Wl2, bl2, Wr2, br2, att2, bo2):
    raise NotImplementedError("write your pallas kernel here")


# speedup vs baseline: 10.5555x; 10.5555x over previous
import functools

import jax
import jax.numpy as jnp
from jax import lax
from jax.experimental import pallas as pl
from jax.experimental.pallas import tpu as pltpu
from jax.experimental.pallas import tpu_sc as plsc

N_NODES_K = 10000
NPAD = 10240
E_RAW = 320000
E_SELF = E_RAW + N_NODES_K

NC = 2
NS = 16
L = 16

KDMA = 128
NB = 162
T_EDGES = NB * KDMA
EPAD = T_EDGES * NS

SHIFT = 30.0
HEADS_K = 6
HID_K = 32
HC_K = HEADS_K * HID_K
D_IN_K = 128
D_OUT_K = 128


def _make_matmul_pack01(d_in, blk=1024):
    H, C = HEADS_K, HID_K
    HC = H * C
    HW = HC // 2

    def body(h_ref, wl_ref, bl_ref, wr_ref, br_ref, att_ref, rw_ref, rb_ref,
             xl_ref, xrm_ref, res_ref):
        hb = h_ref[...]
        xl = jnp.dot(hb, wl_ref[...], preferred_element_type=jnp.float32) + bl_ref[...]
        xr = jnp.dot(hb, wr_ref[...], preferred_element_type=jnp.float32) + br_ref[...]
        s = (xl + xr).reshape(blk, H, C)
        lf = jnp.where(s > 0, s, 0.2 * s)
        aself = jnp.sum(lf * att_ref[...][None], axis=-1) + SHIFT
        pad = jnp.zeros((blk, 13), jnp.float32)
        xl_ref[...] = jnp.stack([xl[:, :HW], xl[:, HW:]], axis=0)
        xrm_ref[...] = jnp.stack([
            jnp.concatenate([xr[:, :HW], aself[:, :3], pad], axis=1),
            jnp.concatenate([xr[:, HW:], aself[:, 3:], pad], axis=1),
        ], axis=0)
        res_ref[...] = (
            jnp.dot(hb, rw_ref[...], preferred_element_type=jnp.float32)
            + rb_ref[...]
        )

    grid = NPAD // blk
    return pl.pallas_call(
        body,
        grid=(grid,),
        in_specs=[
            pl.BlockSpec((blk, d_in), lambda i: (i, 0)),
            pl.BlockSpec((d_in, HC), lambda i: (0, 0)),
            pl.BlockSpec((1, HC), lambda i: (0, 0)),
            pl.BlockSpec((d_in, HC), lambda i: (0, 0)),
            pl.BlockSpec((1, HC), lambda i: (0, 0)),
            pl.BlockSpec((H, C), lambda i: (0, 0)),
            pl.BlockSpec((d_in, HC), lambda i: (0, 0)),
            pl.BlockSpec((1, HC), lambda i: (0, 0)),
        ],
        out_specs=[
            pl.BlockSpec((2, blk, HW), lambda i: (0, i, 0)),
            pl.BlockSpec((2, blk, HW + 16), lambda i: (0, i, 0)),
            pl.BlockSpec((blk, HC), lambda i: (i, 0)),
        ],
        out_shape=[
            jax.ShapeDtypeStruct((2, NPAD, HW), jnp.float32),
            jax.ShapeDtypeStruct((2, NPAD, HW + 16), jnp.float32),
            jax.ShapeDtypeStruct((NPAD, HC), jnp.float32),
        ],
    )


def _make_matmul_pack2(d_in, blk=1024):
    C = D_OUT_K

    def body(h_ref, wl_ref, bl_ref, wr_ref, br_ref, att_ref, xl_ref, xrm_ref):
        hb = h_ref[...]
        xl = jnp.dot(hb, wl_ref[...], preferred_element_type=jnp.float32) + bl_ref[...]
        xr = jnp.dot(hb, wr_ref[...], preferred_element_type=jnp.float32) + br_ref[...]
        s = xl + xr
        lf = jnp.where(s > 0, s, 0.2 * s)
        aself = jnp.sum(lf * att_ref[...], axis=-1, keepdims=True) + SHIFT
        pad = jnp.zeros((blk, 15), jnp.float32)
        xl_ref[...] = xl
        xrm_ref[...] = jnp.concatenate([xr, aself, pad], axis=1)

    grid = NPAD // blk
    return pl.pallas_call(
        body,
        grid=(grid,),
        in_specs=[
            pl.BlockSpec((blk, d_in), lambda i: (i, 0)),
            pl.BlockSpec((d_in, C), lambda i: (0, 0)),
            pl.BlockSpec((1, C), lambda i: (0, 0)),
            pl.BlockSpec((d_in, C), lambda i: (0, 0)),
            pl.BlockSpec((1, C), lambda i: (0, 0)),
            pl.BlockSpec((1, C), lambda i: (0, 0)),
        ],
        out_specs=[
            pl.BlockSpec((blk, C), lambda i: (i, 0)),
            pl.BlockSpec((blk, C + 16), lambda i: (i, 0)),
        ],
        out_shape=[
            jax.ShapeDtypeStruct((NPAD, C), jnp.float32),
            jax.ShapeDtypeStruct((NPAD, C + 16), jnp.float32),
        ],
    )


def _zero_ref(ref, nrows, width):
    zeros16 = jnp.zeros((L,), jnp.float32)
    nchunk = width // 16 + (1 if width % 16 else 0)
    for r in range(nrows):
        for t in range(nchunk):
            off = min(t * 16, width - 16)
            ref[r, pl.ds(off, 16)] = zeros16


def _make_gat_edge_sc01():
    H, C = 3, HID_K
    HW = H * C
    WR = HW + 16
    WO = HW + 8
    mesh = plsc.VectorSubcoreMesh(core_axis_name="c", subcore_axis_name="s")
    rows_per_tile = NPAD // NS

    @functools.partial(
        pl.kernel,
        out_type=jax.ShapeDtypeStruct((NC, NPAD, WO), jnp.float32),
        mesh=mesh,
        compiler_params=pltpu.CompilerParams(
            use_tc_tiling_on_sc=False, needs_layout_passes=False),
        scratch_types=[
            pltpu.VMEM((1, KDMA), jnp.int32),
            pltpu.VMEM((1, KDMA), jnp.int32),
            pltpu.VMEM((KDMA, HW), jnp.float32),
            pltpu.VMEM((KDMA, WR), jnp.float32),
            pltpu.VMEM((KDMA, WO), jnp.float32),
            pltpu.VMEM((16, WO), jnp.float32),
            pltpu.VMEM((HEADS_K, C), jnp.float32),
            pltpu.VMEM_SHARED((NPAD, WO), jnp.float32),
        ],
    )
    def k(xl_hbm, xrm_hbm, src_hbm, dst_hbm, att_hbm, out_hbm,
          src_v, dst_v, xj_v, xim_v, ob_v, zb_v, att_v, table):
        cid = lax.axis_index("c")
        sid = lax.axis_index("s")

        _zero_ref(zb_v, 16, WO)
        _zero_ref(ob_v, KDMA, WO)
        for t in range(rows_per_tile // 16):
            pltpu.sync_copy(zb_v, table.at[pl.ds(sid * rows_per_tile + t * 16, 16)])

        pltpu.sync_copy(att_hbm, att_v)

        plsc.subcore_barrier()

        @pl.loop(0, NB)
        def _batch(b):
            pltpu.sync_copy(src_hbm.at[sid, b], src_v.at[0])
            pltpu.sync_copy(dst_hbm.at[sid, b], dst_v.at[0])
            pltpu.sync_copy(xl_hbm.at[cid].at[src_v.at[0]], xj_v)
            pltpu.sync_copy(xrm_hbm.at[cid].at[dst_v.at[0]], xim_v)

            @pl.loop(0, KDMA // L)
            def _sub(sb):
                lanes = lax.iota(jnp.int32, L) + sb * L
                zeros_i = jnp.zeros((L,), jnp.int32)
                for h in range(H):
                    acc = jnp.zeros((L,), jnp.float32)
                    arow = zeros_i + (cid * H + h)
                    xs = []
                    for c in range(C):
                        col = jnp.full((L,), h * C + c, jnp.int32)
                        xjc = plsc.load_gather(xj_v, [lanes, col])
                        xic = plsc.load_gather(xim_v, [lanes, col])
                        attc = plsc.load_gather(
                            att_v, [arow, jnp.full((L,), c, jnp.int32)])
                        z = xjc + xic
                        lf = jnp.maximum(z, 0.2 * z)
                        acc = acc + attc * lf
                        xs.append(xjc)
                    mh = plsc.load_gather(
                        xim_v, [lanes, jnp.full((L,), HW + h, jnp.int32)])
                    ea = jnp.exp(acc - mh)
                    plsc.store_scatter(
                        ob_v, [lanes, jnp.full((L,), HW + h, jnp.int32)], ea)
                    for c in range(C):
                        col = jnp.full((L,), h * C + c, jnp.int32)
                        plsc.store_scatter(ob_v, [lanes, col], xs[c] * ea)

            pltpu.sync_copy(ob_v, table.at[dst_v.at[0]], add=True)

        plsc.subcore_barrier()

        pltpu.sync_copy(
            table.at[pl.ds(sid * rows_per_tile, rows_per_tile)],
            out_hbm.at[cid, pl.ds(sid * rows_per_tile, rows_per_tile)])

    return k


def _make_gat_edge_sc2():
    C = D_OUT_K
    CW = C // 2
    WR = C + 16
    WO = CW + 8
    mesh = plsc.VectorSubcoreMesh(core_axis_name="c", subcore_axis_name="s")
    rows_per_tile = NPAD // NS

    @functools.partial(
        pl.kernel,
        out_type=jax.ShapeDtypeStruct((NC, NPAD, WO), jnp.float32),
        mesh=mesh,
        compiler_params=pltpu.CompilerParams(
            use_tc_tiling_on_sc=False, needs_layout_passes=False),
        scratch_types=[
            pltpu.VMEM((1, KDMA), jnp.int32),
            pltpu.VMEM((1, KDMA), jnp.int32),
            pltpu.VMEM((KDMA, C), jnp.float32),
            pltpu.VMEM((KDMA, WR), jnp.float32),
            pltpu.VMEM((KDMA, WO), jnp.float32),
            pltpu.VMEM((16, WO), jnp.float32),
            pltpu.VMEM((1, C + 16), jnp.float32),
            pltpu.VMEM_SHARED((NPAD, WO), jnp.float32),
        ],
    )
    def k(xl_hbm, xrm_hbm, src_hbm, dst_hbm, att_hbm, out_hbm,
          src_v, dst_v, xj_v, xim_v, ob_v, zb_v, att_v, table):
        cid = lax.axis_index("c")
        sid = lax.axis_index("s")

        _zero_ref(zb_v, 16, WO)
        _zero_ref(ob_v, KDMA, WO)
        for t in range(rows_per_tile // 16):
            pltpu.sync_copy(zb_v, table.at[pl.ds(sid * rows_per_tile + t * 16, 16)])

        pltpu.sync_copy(att_hbm, att_v)

        plsc.subcore_barrier()

        @pl.loop(0, NB)
        def _batch(b):
            pltpu.sync_copy(src_hbm.at[sid, b], src_v.at[0])
            pltpu.sync_copy(dst_hbm.at[sid, b], dst_v.at[0])
            pltpu.sync_copy(xl_hbm.at[src_v.at[0]], xj_v)
            pltpu.sync_copy(xrm_hbm.at[dst_v.at[0]], xim_v)

            @pl.loop(0, KDMA // L)
            def _sub(sb):
                lanes = lax.iota(jnp.int32, L) + sb * L
                zeros_i = jnp.zeros((L,), jnp.int32)
                acc = jnp.zeros((L,), jnp.float32)
                for c in range(C):
                    col = jnp.full((L,), c, jnp.int32)
                    xjc = plsc.load_gather(xj_v, [lanes, col])
                    xic = plsc.load_gather(xim_v, [lanes, col])
                    attc = plsc.load_gather(
                        att_v, [zeros_i, jnp.full((L,), c + 16, jnp.int32)])
                    z = xjc + xic
                    lf = jnp.maximum(z, 0.2 * z)
                    acc = acc + attc * lf
                mh = plsc.load_gather(
                    xim_v, [lanes, jnp.full((L,), C, jnp.int32)])
                ea = jnp.exp(acc - mh)
                plsc.store_scatter(
                    ob_v, [lanes, jnp.full((L,), CW, jnp.int32)], ea)
                for c in range(CW):
                    col = jnp.full((L,), c, jnp.int32) + cid * CW
                    xjc = plsc.load_gather(xj_v, [lanes, col])
                    plsc.store_scatter(
                        ob_v, [lanes, jnp.full((L,), c, jnp.int32)], xjc * ea)

            pltpu.sync_copy(ob_v, table.at[dst_v.at[0]], add=True)

        plsc.subcore_barrier()

        pltpu.sync_copy(
            table.at[pl.ds(sid * rows_per_tile, rows_per_tile)],
            out_hbm.at[cid, pl.ds(sid * rows_per_tile, rows_per_tile)])

    return k


def _make_epilogue01(blk=1024):
    H, C = HEADS_K, HID_K
    HC = H * C
    HW = HC // 2
    WO = HW + 8
    grid = NPAD // blk

    def body(parts_ref, res_ref, bo_ref, g_ref, be_ref, h_ref):
        p0 = parts_ref[0]
        p1 = parts_ref[1]
        acc = jnp.concatenate([p0[:, :HW], p1[:, :HW]], axis=1)
        den = jnp.concatenate([p0[:, HW:HW + 3], p1[:, HW:HW + 3]], axis=1)
        o = acc.reshape(blk, H, C) / (den.reshape(blk, H, 1) + 1e-30)
        o = o.reshape(blk, HC) + bo_ref[...]
        t = o + res_ref[...]
        mu = jnp.mean(t, axis=-1, keepdims=True)
        var = jnp.mean((t - mu) ** 2, axis=-1, keepdims=True)
        y = (t - mu) / jnp.sqrt(var + 1e-5) * g_ref[...] + be_ref[...]
        h_ref[...] = jnp.where(y > 0, y, jnp.exp(y) - 1.0)

    return pl.pallas_call(
        body,
        grid=(grid,),
        in_specs=[
            pl.BlockSpec((NC, blk, WO), lambda i: (0, i, 0)),
            pl.BlockSpec((blk, HC), lambda i: (i, 0)),
            pl.BlockSpec((1, HC), lambda i: (0, 0)),
            pl.BlockSpec((1, HC), lambda i: (0, 0)),
            pl.BlockSpec((1, HC), lambda i: (0, 0)),
        ],
        out_specs=pl.BlockSpec((blk, HC), lambda i: (i, 0)),
        out_shape=jax.ShapeDtypeStruct((NPAD, HC), jnp.float32),
    )


def _make_final_epilogue(blk=1024):
    C = D_OUT_K
    CW = C // 2
    WO = CW + 8
    grid = NPAD // blk

    def body(parts_ref, bo_ref, out_ref):
        p0 = parts_ref[0]
        p1 = parts_ref[1]
        o0 = p0[:, :CW] / (p0[:, CW:CW + 1] + 1e-30)
        o1 = p1[:, :CW] / (p1[:, CW:CW + 1] + 1e-30)
        out_ref[...] = jnp.concatenate([o0, o1], axis=1) + bo_ref[...]

    return pl.pallas_call(
        body,
        grid=(grid,),
        in_specs=[
            pl.BlockSpec((NC, blk, WO), lambda i: (0, i, 0)),
            pl.BlockSpec((1, C), lambda i: (0, 0)),
        ],
        out_specs=pl.BlockSpec((blk, C), lambda i: (i, 0)),
        out_shape=jax.ShapeDtypeStruct((NPAD, C), jnp.float32),
    )


_mm0 = _make_matmul_pack01(D_IN_K)
_mm1 = _make_matmul_pack01(HC_K)
_mm2 = _make_matmul_pack2(HC_K)
_sc01 = _make_gat_edge_sc01()
_sc2 = _make_gat_edge_sc2()
_epi01 = _make_epilogue01()
_epi2 = _make_final_epilogue()


def kernel(x, edge_index, Wl0, bl0, Wr0, br0, att0, bo0, rW0, rb0, g0, be0,
           Wl1, bl1, Wr1, br1, att1, bo1, rW1, rb1, g1, be1,
           Wl2, bl2, Wr2, br2, att2, bo2):
    n = x.shape[0]
    si = jnp.arange(n, dtype=jnp.int32)
    pad_ids = jnp.full((EPAD - E_SELF,), n, jnp.int32)
    src2 = jnp.concatenate([edge_index[0], si, pad_ids]).reshape(NS, NB, KDMA)
    dst2 = jnp.concatenate([edge_index[1], si, pad_ids]).reshape(NS, NB, KDMA)

    xp = jnp.pad(x, ((0, NPAD - n), (0, 0)))

    xl, xrm, res = _mm0(xp, Wl0, bl0.reshape(1, -1), Wr0, br0.reshape(1, -1),
                        att0, rW0, rb0.reshape(1, -1))
    parts = _sc01(xl, xrm, src2, dst2, att0)
    h = _epi01(parts, res, bo0.reshape(1, -1), g0.reshape(1, -1),
               be0.reshape(1, -1))

    xl, xrm, res = _mm1(h, Wl1, bl1.reshape(1, -1), Wr1, br1.reshape(1, -1),
                        att1, rW1, rb1.reshape(1, -1))
    parts = _sc01(xl, xrm, src2, dst2, att1)
    h = _epi01(parts, res, bo1.reshape(1, -1), g1.reshape(1, -1),
               be1.reshape(1, -1))

    xl, xrm = _mm2(h, Wl2, bl2.reshape(1, -1), Wr2, br2.reshape(1, -1), att2)
    att2p = jnp.pad(att2, ((0, 0), (16, 0)))
    parts = _sc2(xl, xrm, src2, dst2, att2p)
    out = _epi2(parts, bo2.reshape(1, -1))
    return out[:n]


# speedup vs baseline: 13.6171x; 1.2901x over previous
import functools

import jax
import jax.numpy as jnp
from jax import lax
from jax.experimental import pallas as pl
from jax.experimental.pallas import tpu as pltpu
from jax.experimental.pallas import tpu_sc as plsc

N_NODES_K = 10000
NPAD = 10240
E_RAW = 320000
E_SELF = E_RAW + N_NODES_K

NC = 2
NS = 16
L = 16

KDMA = 64
NB = 324
CHUNK = 12
NCHUNK = NB // CHUNK
T_EDGES = NB * KDMA
EPAD = T_EDGES * NS

SHIFT = 30.0
HEADS_K = 6
HID_K = 32
HC_K = HEADS_K * HID_K
D_IN_K = 128
D_OUT_K = 128


def _make_matmul_pack01(d_in, blk=1024):
    H, C = HEADS_K, HID_K
    HC = H * C
    HW = HC // 2

    def body(h_ref, wl_ref, bl_ref, wr_ref, br_ref, att_ref, rw_ref, rb_ref,
             xl_ref, xrm_ref, res_ref):
        hb = h_ref[...]
        xl = jnp.dot(hb, wl_ref[...], preferred_element_type=jnp.float32) + bl_ref[...]
        xr = jnp.dot(hb, wr_ref[...], preferred_element_type=jnp.float32) + br_ref[...]
        s = (xl + xr).reshape(blk, H, C)
        lf = jnp.where(s > 0, s, 0.2 * s)
        aself = jnp.sum(lf * att_ref[...][None], axis=-1) + SHIFT
        pad = jnp.zeros((blk, 13), jnp.float32)
        xl_ref[...] = jnp.stack([xl[:, :HW], xl[:, HW:]], axis=0)
        xrm_ref[...] = jnp.stack([
            jnp.concatenate([xr[:, :HW], aself[:, :3], pad], axis=1),
            jnp.concatenate([xr[:, HW:], aself[:, 3:], pad], axis=1),
        ], axis=0)
        res_ref[...] = (
            jnp.dot(hb, rw_ref[...], preferred_element_type=jnp.float32)
            + rb_ref[...]
        )

    grid = NPAD // blk
    return pl.pallas_call(
        body,
        grid=(grid,),
        in_specs=[
            pl.BlockSpec((blk, d_in), lambda i: (i, 0)),
            pl.BlockSpec((d_in, HC), lambda i: (0, 0)),
            pl.BlockSpec((1, HC), lambda i: (0, 0)),
            pl.BlockSpec((d_in, HC), lambda i: (0, 0)),
            pl.BlockSpec((1, HC), lambda i: (0, 0)),
            pl.BlockSpec((H, C), lambda i: (0, 0)),
            pl.BlockSpec((d_in, HC), lambda i: (0, 0)),
            pl.BlockSpec((1, HC), lambda i: (0, 0)),
        ],
        out_specs=[
            pl.BlockSpec((2, blk, HW), lambda i: (0, i, 0)),
            pl.BlockSpec((2, blk, HW + 16), lambda i: (0, i, 0)),
            pl.BlockSpec((blk, HC), lambda i: (i, 0)),
        ],
        out_shape=[
            jax.ShapeDtypeStruct((2, NPAD, HW), jnp.float32),
            jax.ShapeDtypeStruct((2, NPAD, HW + 16), jnp.float32),
            jax.ShapeDtypeStruct((NPAD, HC), jnp.float32),
        ],
    )


def _make_matmul_pack2(d_in, blk=1024):
    C = D_OUT_K

    def body(h_ref, wl_ref, bl_ref, wr_ref, br_ref, att_ref, xl_ref, xrm_ref):
        hb = h_ref[...]
        xl = jnp.dot(hb, wl_ref[...], preferred_element_type=jnp.float32) + bl_ref[...]
        xr = jnp.dot(hb, wr_ref[...], preferred_element_type=jnp.float32) + br_ref[...]
        s = xl + xr
        lf = jnp.where(s > 0, s, 0.2 * s)
        aself = jnp.sum(lf * att_ref[...], axis=-1, keepdims=True) + SHIFT
        pad = jnp.zeros((blk, 15), jnp.float32)
        xl_ref[...] = xl
        xrm_ref[...] = jnp.concatenate([xr, aself, pad], axis=1)

    grid = NPAD // blk
    return pl.pallas_call(
        body,
        grid=(grid,),
        in_specs=[
            pl.BlockSpec((blk, d_in), lambda i: (i, 0)),
            pl.BlockSpec((d_in, C), lambda i: (0, 0)),
            pl.BlockSpec((1, C), lambda i: (0, 0)),
            pl.BlockSpec((d_in, C), lambda i: (0, 0)),
            pl.BlockSpec((1, C), lambda i: (0, 0)),
            pl.BlockSpec((1, C), lambda i: (0, 0)),
        ],
        out_specs=[
            pl.BlockSpec((blk, C), lambda i: (i, 0)),
            pl.BlockSpec((blk, C + 16), lambda i: (i, 0)),
        ],
        out_shape=[
            jax.ShapeDtypeStruct((NPAD, C), jnp.float32),
            jax.ShapeDtypeStruct((NPAD, C + 16), jnp.float32),
        ],
    )


def _zero_ref(ref, nrows, width):
    zeros16 = jnp.zeros((L,), jnp.float32)
    nchunk = width // 16 + (1 if width % 16 else 0)
    for r in range(nrows):
        for t in range(nchunk):
            off = min(t * 16, width - 16)
            ref[r, pl.ds(off, 16)] = zeros16


def _make_gat_edge_sc01():
    H, C = 3, HID_K
    HW = H * C
    WR = HW + 16
    WO = HW + 8
    mesh = plsc.VectorSubcoreMesh(core_axis_name="c", subcore_axis_name="s")
    rows_per_tile = NPAD // NS

    @functools.partial(
        pl.kernel,
        out_type=jax.ShapeDtypeStruct((NC, NPAD, WO), jnp.float32),
        mesh=mesh,
        compiler_params=pltpu.CompilerParams(
            use_tc_tiling_on_sc=False, needs_layout_passes=False),
        scratch_types=[
            pltpu.VMEM((2, CHUNK, KDMA), jnp.int32),
            pltpu.VMEM((2, CHUNK, KDMA), jnp.int32),
            pltpu.VMEM((2, KDMA, HW), jnp.float32),
            pltpu.VMEM((2, KDMA, WR), jnp.float32),
            pltpu.VMEM((2, KDMA, WO), jnp.float32),
            pltpu.VMEM((16, WO), jnp.float32),
            pltpu.VMEM((HEADS_K, C), jnp.float32),
            pltpu.VMEM_SHARED((NPAD, WO), jnp.float32),
            pltpu.SemaphoreType.DMA,
            pltpu.SemaphoreType.DMA,
            pltpu.SemaphoreType.DMA,
            pltpu.SemaphoreType.DMA,
            pltpu.SemaphoreType.DMA,
            pltpu.SemaphoreType.DMA,
        ],
    )
    def k(xl_hbm, xrm_hbm, src_hbm, dst_hbm, att_hbm, out_hbm,
          src_v, dst_v, xj_v, xim_v, ob_v, zb_v, att_v, table,
          sgx0, sgx1, sgm0, sgm1, ssc0, ssc1):
        cid = lax.axis_index("c")
        sid = lax.axis_index("s")
        sgx = (sgx0, sgx1)
        sgm = (sgm0, sgm1)
        ssc = (ssc0, ssc1)

        _zero_ref(zb_v, 16, WO)
        _zero_ref(ob_v.at[0], KDMA, WO)
        _zero_ref(ob_v.at[1], KDMA, WO)
        for t in range(rows_per_tile // 16):
            pltpu.sync_copy(zb_v, table.at[pl.ds(sid * rows_per_tile + t * 16, 16)])

        pltpu.sync_copy(att_hbm, att_v)

        plsc.subcore_barrier()

        pltpu.sync_copy(src_hbm.at[sid, 0], src_v.at[0])
        pltpu.sync_copy(dst_hbm.at[sid, 0], dst_v.at[0])
        pltpu.async_copy(xl_hbm.at[cid].at[src_v.at[0, 0]], xj_v.at[0], sgx[0])
        pltpu.async_copy(xrm_hbm.at[cid].at[dst_v.at[0, 0]], xim_v.at[0], sgm[0])

        def compute(par, xjr, ximr, obr):
            @pl.loop(0, KDMA // L)
            def _sub(sb):
                lanes = lax.iota(jnp.int32, L) + sb * L
                zeros_i = jnp.zeros((L,), jnp.int32)
                for h in range(H):
                    acc = jnp.zeros((L,), jnp.float32)
                    arow = zeros_i + (cid * H + h)
                    xs = []
                    for c in range(C):
                        col = jnp.full((L,), h * C + c, jnp.int32)
                        xjc = plsc.load_gather(xjr, [lanes, col])
                        xic = plsc.load_gather(ximr, [lanes, col])
                        attc = plsc.load_gather(
                            att_v, [arow, jnp.full((L,), c, jnp.int32)])
                        z = xjc + xic
                        lf = jnp.maximum(z, 0.2 * z)
                        acc = acc + attc * lf
                        xs.append(xjc)
                    mh = plsc.load_gather(
                        ximr, [lanes, jnp.full((L,), HW + h, jnp.int32)])
                    ea = jnp.exp(acc - mh)
                    plsc.store_scatter(
                        obr, [lanes, jnp.full((L,), HW + h, jnp.int32)], ea)
                    for c in range(C):
                        col = jnp.full((L,), h * C + c, jnp.int32)
                        plsc.store_scatter(obr, [lanes, col], xs[c] * ea)

        @pl.loop(0, NB, step=2)
        def _pair(i):
            for par in range(2):
                b = i + par
                cp = (b // CHUNK) % 2
                jr = b % CHUNK
                bn = b + 1
                cn = bn // CHUNK
                jn = bn % CHUNK
                cpn = cn % 2
                pltpu.make_async_copy(
                    xl_hbm.at[cid].at[src_v.at[cp, jr]],
                    xj_v.at[par], sgx[par]).wait()
                pltpu.make_async_copy(
                    xrm_hbm.at[cid].at[dst_v.at[cp, jr]],
                    xim_v.at[par], sgm[par]).wait()

                @pl.when(jn == 0)
                def _():
                    pltpu.sync_copy(src_hbm.at[sid, cn], src_v.at[cpn])
                    pltpu.sync_copy(dst_hbm.at[sid, cn], dst_v.at[cpn])

                pltpu.async_copy(xl_hbm.at[cid].at[src_v.at[cpn, jn]],
                                 xj_v.at[1 - par], sgx[1 - par])
                pltpu.async_copy(xrm_hbm.at[cid].at[dst_v.at[cpn, jn]],
                                 xim_v.at[1 - par], sgm[1 - par])

                @pl.when(b >= 2)
                def _():
                    pltpu.make_async_copy(
                        ob_v.at[par], table.at[dst_v.at[cp, jr]],
                        ssc[par]).wait()

                compute(par, xj_v.at[par], xim_v.at[par], ob_v.at[par])

                pltpu.async_copy(ob_v.at[par], table.at[dst_v.at[cp, jr]],
                                 ssc[par], add=True)

        lb = NB - 2
        for par in range(2):
            b = lb + par
            cp = (b // CHUNK) % 2
            jr = b % CHUNK
            pltpu.make_async_copy(
                ob_v.at[par], table.at[dst_v.at[cp, jr]], ssc[par]).wait()
        cpo = (NB // CHUNK) % 2
        pltpu.make_async_copy(
            xl_hbm.at[cid].at[src_v.at[cpo, 0]], xj_v.at[0], sgx[0]).wait()
        pltpu.make_async_copy(
            xrm_hbm.at[cid].at[dst_v.at[cpo, 0]], xim_v.at[0], sgm[0]).wait()

        plsc.subcore_barrier()

        pltpu.sync_copy(
            table.at[pl.ds(sid * rows_per_tile, rows_per_tile)],
            out_hbm.at[cid, pl.ds(sid * rows_per_tile, rows_per_tile)])

    return k


def _make_gat_edge_sc2():
    C = D_OUT_K
    CW = C // 2
    WR = C + 16
    WO = CW + 8
    mesh = plsc.VectorSubcoreMesh(core_axis_name="c", subcore_axis_name="s")
    rows_per_tile = NPAD // NS

    @functools.partial(
        pl.kernel,
        out_type=jax.ShapeDtypeStruct((NC, NPAD, WO), jnp.float32),
        mesh=mesh,
        compiler_params=pltpu.CompilerParams(
            use_tc_tiling_on_sc=False, needs_layout_passes=False),
        scratch_types=[
            pltpu.VMEM((2, CHUNK, KDMA), jnp.int32),
            pltpu.VMEM((2, CHUNK, KDMA), jnp.int32),
            pltpu.VMEM((2, KDMA, C), jnp.float32),
            pltpu.VMEM((2, KDMA, WR), jnp.float32),
            pltpu.VMEM((2, KDMA, WO), jnp.float32),
            pltpu.VMEM((16, WO), jnp.float32),
            pltpu.VMEM((1, C + 16), jnp.float32),
            pltpu.VMEM_SHARED((NPAD, WO), jnp.float32),
            pltpu.SemaphoreType.DMA,
            pltpu.SemaphoreType.DMA,
            pltpu.SemaphoreType.DMA,
            pltpu.SemaphoreType.DMA,
            pltpu.SemaphoreType.DMA,
            pltpu.SemaphoreType.DMA,
        ],
    )
    def k(xl_hbm, xrm_hbm, src_hbm, dst_hbm, att_hbm, out_hbm,
          src_v, dst_v, xj_v, xim_v, ob_v, zb_v, att_v, table,
          sgx0, sgx1, sgm0, sgm1, ssc0, ssc1):
        cid = lax.axis_index("c")
        sid = lax.axis_index("s")
        sgx = (sgx0, sgx1)
        sgm = (sgm0, sgm1)
        ssc = (ssc0, ssc1)

        _zero_ref(zb_v, 16, WO)
        _zero_ref(ob_v.at[0], KDMA, WO)
        _zero_ref(ob_v.at[1], KDMA, WO)
        for t in range(rows_per_tile // 16):
            pltpu.sync_copy(zb_v, table.at[pl.ds(sid * rows_per_tile + t * 16, 16)])

        pltpu.sync_copy(att_hbm, att_v)

        plsc.subcore_barrier()

        pltpu.sync_copy(src_hbm.at[sid, 0], src_v.at[0])
        pltpu.sync_copy(dst_hbm.at[sid, 0], dst_v.at[0])
        pltpu.async_copy(xl_hbm.at[src_v.at[0, 0]], xj_v.at[0], sgx[0])
        pltpu.async_copy(xrm_hbm.at[dst_v.at[0, 0]], xim_v.at[0], sgm[0])

        def compute(xjr, ximr, obr):
            @pl.loop(0, KDMA // L)
            def _sub(sb):
                lanes = lax.iota(jnp.int32, L) + sb * L
                zeros_i = jnp.zeros((L,), jnp.int32)
                acc = jnp.zeros((L,), jnp.float32)
                for c in range(C):
                    col = jnp.full((L,), c, jnp.int32)
                    xjc = plsc.load_gather(xjr, [lanes, col])
                    xic = plsc.load_gather(ximr, [lanes, col])
                    attc = plsc.load_gather(
                        att_v, [zeros_i, jnp.full((L,), c + 16, jnp.int32)])
                    z = xjc + xic
                    lf = jnp.maximum(z, 0.2 * z)
                    acc = acc + attc * lf
                mh = plsc.load_gather(
                    ximr, [lanes, jnp.full((L,), C, jnp.int32)])
                ea = jnp.exp(acc - mh)
                plsc.store_scatter(
                    obr, [lanes, jnp.full((L,), CW, jnp.int32)], ea)
                for c in range(CW):
                    col = jnp.full((L,), c, jnp.int32) + cid * CW
                    xjc = plsc.load_gather(xjr, [lanes, col])
                    plsc.store_scatter(
                        obr, [lanes, jnp.full((L,), c, jnp.int32)], xjc * ea)

        @pl.loop(0, NB, step=2)
        def _pair(i):
            for par in range(2):
                b = i + par
                cp = (b // CHUNK) % 2
                jr = b % CHUNK
                bn = b + 1
                cn = bn // CHUNK
                jn = bn % CHUNK
                cpn = cn % 2
                pltpu.make_async_copy(
                    xl_hbm.at[src_v.at[cp, jr]], xj_v.at[par], sgx[par]).wait()
                pltpu.make_async_copy(
                    xrm_hbm.at[dst_v.at[cp, jr]], xim_v.at[par], sgm[par]).wait()

                @pl.when(jn == 0)
                def _():
                    pltpu.sync_copy(src_hbm.at[sid, cn], src_v.at[cpn])
                    pltpu.sync_copy(dst_hbm.at[sid, cn], dst_v.at[cpn])

                pltpu.async_copy(xl_hbm.at[src_v.at[cpn, jn]],
                                 xj_v.at[1 - par], sgx[1 - par])
                pltpu.async_copy(xrm_hbm.at[dst_v.at[cpn, jn]],
                                 xim_v.at[1 - par], sgm[1 - par])

                @pl.when(b >= 2)
                def _():
                    pltpu.make_async_copy(
                        ob_v.at[par], table.at[dst_v.at[cp, jr]],
                        ssc[par]).wait()

                compute(xj_v.at[par], xim_v.at[par], ob_v.at[par])

                pltpu.async_copy(ob_v.at[par], table.at[dst_v.at[cp, jr]],
                                 ssc[par], add=True)

        lb = NB - 2
        for par in range(2):
            b = lb + par
            cp = (b // CHUNK) % 2
            jr = b % CHUNK
            pltpu.make_async_copy(
                ob_v.at[par], table.at[dst_v.at[cp, jr]], ssc[par]).wait()
        cpo = (NB // CHUNK) % 2
        pltpu.make_async_copy(
            xl_hbm.at[src_v.at[cpo, 0]], xj_v.at[0], sgx[0]).wait()
        pltpu.make_async_copy(
            xrm_hbm.at[dst_v.at[cpo, 0]], xim_v.at[0], sgm[0]).wait()

        plsc.subcore_barrier()

        pltpu.sync_copy(
            table.at[pl.ds(sid * rows_per_tile, rows_per_tile)],
            out_hbm.at[cid, pl.ds(sid * rows_per_tile, rows_per_tile)])

    return k


def _make_epilogue01(blk=1024):
    H, C = HEADS_K, HID_K
    HC = H * C
    HW = HC // 2
    WO = HW + 8
    grid = NPAD // blk

    def body(parts_ref, res_ref, bo_ref, g_ref, be_ref, h_ref):
        p0 = parts_ref[0]
        p1 = parts_ref[1]
        acc = jnp.concatenate([p0[:, :HW], p1[:, :HW]], axis=1)
        den = jnp.concatenate([p0[:, HW:HW + 3], p1[:, HW:HW + 3]], axis=1)
        o = acc.reshape(blk, H, C) / (den.reshape(blk, H, 1) + 1e-30)
        o = o.reshape(blk, HC) + bo_ref[...]
        t = o + res_ref[...]
        mu = jnp.mean(t, axis=-1, keepdims=True)
        var = jnp.mean((t - mu) ** 2, axis=-1, keepdims=True)
        y = (t - mu) / jnp.sqrt(var + 1e-5) * g_ref[...] + be_ref[...]
        h_ref[...] = jnp.where(y > 0, y, jnp.exp(y) - 1.0)

    return pl.pallas_call(
        body,
        grid=(grid,),
        in_specs=[
            pl.BlockSpec((NC, blk, WO), lambda i: (0, i, 0)),
            pl.BlockSpec((blk, HC), lambda i: (i, 0)),
            pl.BlockSpec((1, HC), lambda i: (0, 0)),
            pl.BlockSpec((1, HC), lambda i: (0, 0)),
            pl.BlockSpec((1, HC), lambda i: (0, 0)),
        ],
        out_specs=pl.BlockSpec((blk, HC), lambda i: (i, 0)),
        out_shape=jax.ShapeDtypeStruct((NPAD, HC), jnp.float32),
    )


def _make_final_epilogue(blk=1024):
    C = D_OUT_K
    CW = C // 2
    WO = CW + 8
    grid = NPAD // blk

    def body(parts_ref, bo_ref, out_ref):
        p0 = parts_ref[0]
        p1 = parts_ref[1]
        o0 = p0[:, :CW] / (p0[:, CW:CW + 1] + 1e-30)
        o1 = p1[:, :CW] / (p1[:, CW:CW + 1] + 1e-30)
        out_ref[...] = jnp.concatenate([o0, o1], axis=1) + bo_ref[...]

    return pl.pallas_call(
        body,
        grid=(grid,),
        in_specs=[
            pl.BlockSpec((NC, blk, WO), lambda i: (0, i, 0)),
            pl.BlockSpec((1, C), lambda i: (0, 0)),
        ],
        out_specs=pl.BlockSpec((blk, C), lambda i: (i, 0)),
        out_shape=jax.ShapeDtypeStruct((NPAD, C), jnp.float32),
    )


_mm0 = _make_matmul_pack01(D_IN_K)
_mm1 = _make_matmul_pack01(HC_K)
_mm2 = _make_matmul_pack2(HC_K)
_sc01 = _make_gat_edge_sc01()
_sc2 = _make_gat_edge_sc2()
_epi01 = _make_epilogue01()
_epi2 = _make_final_epilogue()


def kernel(x, edge_index, Wl0, bl0, Wr0, br0, att0, bo0, rW0, rb0, g0, be0,
           Wl1, bl1, Wr1, br1, att1, bo1, rW1, rb1, g1, be1,
           Wl2, bl2, Wr2, br2, att2, bo2):
    n = x.shape[0]
    si = jnp.arange(n, dtype=jnp.int32)
    pad_ids = jnp.full((EPAD - E_SELF,), n, jnp.int32)
    dummy = jnp.full((NS, CHUNK * KDMA), n, jnp.int32)

    def _edge_arr(e):
        per_tile = jnp.concatenate([e, si, pad_ids]).reshape(NS, T_EDGES)
        return jnp.concatenate([per_tile, dummy], axis=1).reshape(
            NS, NCHUNK + 1, CHUNK, KDMA)

    src2 = _edge_arr(edge_index[0])
    dst2 = _edge_arr(edge_index[1])

    xp = jnp.pad(x, ((0, NPAD - n), (0, 0)))

    xl, xrm, res = _mm0(xp, Wl0, bl0.reshape(1, -1), Wr0, br0.reshape(1, -1),
                        att0, rW0, rb0.reshape(1, -1))
    parts = _sc01(xl, xrm, src2, dst2, att0)
    h = _epi01(parts, res, bo0.reshape(1, -1), g0.reshape(1, -1),
               be0.reshape(1, -1))

    xl, xrm, res = _mm1(h, Wl1, bl1.reshape(1, -1), Wr1, br1.reshape(1, -1),
                        att1, rW1, rb1.reshape(1, -1))
    parts = _sc01(xl, xrm, src2, dst2, att1)
    h = _epi01(parts, res, bo1.reshape(1, -1), g1.reshape(1, -1),
               be1.reshape(1, -1))

    xl, xrm = _mm2(h, Wl2, bl2.reshape(1, -1), Wr2, br2.reshape(1, -1), att2)
    att2p = jnp.pad(att2, ((0, 0), (16, 0)))
    parts = _sc2(xl, xrm, src2, dst2, att2p)
    out = _epi2(parts, bo2.reshape(1, -1))
    return out[:n]


# speedup vs baseline: 15.8799x; 1.1662x over previous
import functools

import jax
import jax.numpy as jnp
from jax import lax
from jax.experimental import pallas as pl
from jax.experimental.pallas import tpu as pltpu
from jax.experimental.pallas import tpu_sc as plsc

N_NODES_K = 10000
NPAD = 10240
E_RAW = 320000
E_SELF = E_RAW + N_NODES_K

NC = 2
NS = 16
L = 16

KDMA = 64
NB = 324
CHUNK = 12
NCHUNK = NB // CHUNK
T_EDGES = NB * KDMA
EPAD = T_EDGES * NS

SHIFT = 30.0
HEADS_K = 6
HID_K = 32
HC_K = HEADS_K * HID_K
D_IN_K = 128
D_OUT_K = 128


def _make_matmul_pack01(d_in, blk=1024):
    H, C = HEADS_K, HID_K
    HC = H * C
    HW = HC // 2

    def body(h_ref, wl_ref, bl_ref, wr_ref, br_ref, att_ref, rw_ref, rb_ref,
             xl_ref, xrm_ref, res_ref):
        hb = h_ref[...]
        xl = jnp.dot(hb, wl_ref[...], preferred_element_type=jnp.float32) + bl_ref[...]
        xr = jnp.dot(hb, wr_ref[...], preferred_element_type=jnp.float32) + br_ref[...]
        s = (xl + xr).reshape(blk, H, C)
        lf = jnp.where(s > 0, s, 0.2 * s)
        aself = jnp.sum(lf * att_ref[...][None], axis=-1) + SHIFT
        pad = jnp.zeros((blk, 13), jnp.float32)
        xl_ref[...] = jnp.stack([xl[:, :HW], xl[:, HW:]], axis=0)
        xrm_ref[...] = jnp.stack([
            jnp.concatenate([xr[:, :HW], aself[:, :3], pad], axis=1),
            jnp.concatenate([xr[:, HW:], aself[:, 3:], pad], axis=1),
        ], axis=0)
        res_ref[...] = (
            jnp.dot(hb, rw_ref[...], preferred_element_type=jnp.float32)
            + rb_ref[...]
        )

    grid = NPAD // blk
    return pl.pallas_call(
        body,
        grid=(grid,),
        in_specs=[
            pl.BlockSpec((blk, d_in), lambda i: (i, 0)),
            pl.BlockSpec((d_in, HC), lambda i: (0, 0)),
            pl.BlockSpec((1, HC), lambda i: (0, 0)),
            pl.BlockSpec((d_in, HC), lambda i: (0, 0)),
            pl.BlockSpec((1, HC), lambda i: (0, 0)),
            pl.BlockSpec((H, C), lambda i: (0, 0)),
            pl.BlockSpec((d_in, HC), lambda i: (0, 0)),
            pl.BlockSpec((1, HC), lambda i: (0, 0)),
        ],
        out_specs=[
            pl.BlockSpec((2, blk, HW), lambda i: (0, i, 0)),
            pl.BlockSpec((2, blk, HW + 16), lambda i: (0, i, 0)),
            pl.BlockSpec((blk, HC), lambda i: (i, 0)),
        ],
        out_shape=[
            jax.ShapeDtypeStruct((2, NPAD, HW), jnp.float32),
            jax.ShapeDtypeStruct((2, NPAD, HW + 16), jnp.float32),
            jax.ShapeDtypeStruct((NPAD, HC), jnp.float32),
        ],
    )


def _make_matmul_pack2(d_in, blk=1024):
    C = D_OUT_K

    def body(h_ref, wl_ref, bl_ref, wr_ref, br_ref, att_ref, xl_ref, xrm_ref):
        hb = h_ref[...]
        xl = jnp.dot(hb, wl_ref[...], preferred_element_type=jnp.float32) + bl_ref[...]
        xr = jnp.dot(hb, wr_ref[...], preferred_element_type=jnp.float32) + br_ref[...]
        s = xl + xr
        lf = jnp.where(s > 0, s, 0.2 * s)
        aself = jnp.sum(lf * att_ref[...], axis=-1, keepdims=True) + SHIFT
        pad = jnp.zeros((blk, 15), jnp.float32)
        xl_ref[...] = xl
        xrm_ref[...] = jnp.concatenate([xr, aself, pad], axis=1)

    grid = NPAD // blk
    return pl.pallas_call(
        body,
        grid=(grid,),
        in_specs=[
            pl.BlockSpec((blk, d_in), lambda i: (i, 0)),
            pl.BlockSpec((d_in, C), lambda i: (0, 0)),
            pl.BlockSpec((1, C), lambda i: (0, 0)),
            pl.BlockSpec((d_in, C), lambda i: (0, 0)),
            pl.BlockSpec((1, C), lambda i: (0, 0)),
            pl.BlockSpec((1, C), lambda i: (0, 0)),
        ],
        out_specs=[
            pl.BlockSpec((blk, C), lambda i: (i, 0)),
            pl.BlockSpec((blk, C + 16), lambda i: (i, 0)),
        ],
        out_shape=[
            jax.ShapeDtypeStruct((NPAD, C), jnp.float32),
            jax.ShapeDtypeStruct((NPAD, C + 16), jnp.float32),
        ],
    )


def _zero_ref(ref, nrows, width):
    zeros16 = jnp.zeros((L,), jnp.float32)
    nchunk = width // 16 + (1 if width % 16 else 0)
    for r in range(nrows):
        for t in range(nchunk):
            off = min(t * 16, width - 16)
            ref[r, pl.ds(off, 16)] = zeros16


def _make_gat_edge_sc01():
    H, C = 3, HID_K
    HW = H * C
    WR = HW + 16
    WO = HW + 8
    mesh = plsc.VectorSubcoreMesh(core_axis_name="c", subcore_axis_name="s")
    rows_per_tile = NPAD // NS

    @functools.partial(
        pl.kernel,
        out_type=jax.ShapeDtypeStruct((NC, NPAD, WO), jnp.float32),
        mesh=mesh,
        compiler_params=pltpu.CompilerParams(
            use_tc_tiling_on_sc=False, needs_layout_passes=False),
        scratch_types=[
            pltpu.VMEM((2, CHUNK, KDMA), jnp.int32),
            pltpu.VMEM((2, CHUNK, KDMA), jnp.int32),
            pltpu.VMEM((2, KDMA, HW), jnp.float32),
            pltpu.VMEM((2, KDMA, WR), jnp.float32),
            pltpu.VMEM((2, KDMA, WO), jnp.float32),
            pltpu.VMEM((16, WO), jnp.float32),
            pltpu.VMEM((HEADS_K, C), jnp.float32),
            pltpu.VMEM_SHARED((NPAD, WO), jnp.float32),
            pltpu.SemaphoreType.DMA,
            pltpu.SemaphoreType.DMA,
            pltpu.SemaphoreType.DMA,
            pltpu.SemaphoreType.DMA,
            pltpu.SemaphoreType.DMA,
            pltpu.SemaphoreType.DMA,
        ],
    )
    def k(xl_hbm, xrm_hbm, src_hbm, dst_hbm, att_hbm, out_hbm,
          src_v, dst_v, xj_v, xim_v, ob_v, zb_v, att_v, table,
          sgx0, sgx1, sgm0, sgm1, ssc0, ssc1):
        cid = lax.axis_index("c")
        sid = lax.axis_index("s")
        sgx = (sgx0, sgx1)
        sgm = (sgm0, sgm1)
        ssc = (ssc0, ssc1)

        _zero_ref(zb_v, 16, WO)
        _zero_ref(ob_v.at[0], KDMA, WO)
        _zero_ref(ob_v.at[1], KDMA, WO)
        for t in range(rows_per_tile // 16):
            pltpu.sync_copy(zb_v, table.at[pl.ds(sid * rows_per_tile + t * 16, 16)])

        pltpu.sync_copy(att_hbm, att_v)

        plsc.subcore_barrier()

        pltpu.sync_copy(src_hbm.at[sid, 0], src_v.at[0])
        pltpu.sync_copy(dst_hbm.at[sid, 0], dst_v.at[0])
        pltpu.async_copy(xl_hbm.at[cid].at[src_v.at[0, 0]], xj_v.at[0], sgx[0])
        pltpu.async_copy(xrm_hbm.at[cid].at[dst_v.at[0, 0]], xim_v.at[0], sgm[0])

        def compute(par, xjr, ximr, obr):
            @pl.loop(0, KDMA // L)
            def _sub(sb):
                lanes = lax.iota(jnp.int32, L) + sb * L
                zeros_i = jnp.zeros((L,), jnp.int32)
                for h in range(H):
                    acc = jnp.zeros((L,), jnp.float32)
                    arow = zeros_i + (cid * H + h)
                    xs = []
                    for c in range(C):
                        col = jnp.full((L,), h * C + c, jnp.int32)
                        xjc = plsc.load_gather(xjr, [lanes, col])
                        xic = plsc.load_gather(ximr, [lanes, col])
                        attc = plsc.load_gather(
                            att_v, [arow, jnp.full((L,), c, jnp.int32)])
                        z = xjc + xic
                        lf = jnp.maximum(z, 0.2 * z)
                        acc = acc + attc * lf
                        xs.append(xjc)
                    mh = plsc.load_gather(
                        ximr, [lanes, jnp.full((L,), HW + h, jnp.int32)])
                    ea = jnp.exp(acc - mh)
                    plsc.store_scatter(
                        obr, [lanes, jnp.full((L,), HW + h, jnp.int32)], ea)
                    for c in range(C):
                        col = jnp.full((L,), h * C + c, jnp.int32)
                        plsc.store_scatter(obr, [lanes, col], xs[c] * ea)

        @pl.loop(0, NB, step=2)
        def _pair(i):
            for par in range(2):
                b = i + par
                cp = (b // CHUNK) % 2
                jr = b % CHUNK
                bn = b + 1
                cn = bn // CHUNK
                jn = bn % CHUNK
                cpn = cn % 2
                pltpu.make_async_copy(
                    xl_hbm.at[cid].at[src_v.at[cp, jr]],
                    xj_v.at[par], sgx[par]).wait()
                pltpu.make_async_copy(
                    xrm_hbm.at[cid].at[dst_v.at[cp, jr]],
                    xim_v.at[par], sgm[par]).wait()

                @pl.when(jn == 0)
                def _():
                    pltpu.sync_copy(src_hbm.at[sid, cn], src_v.at[cpn])
                    pltpu.sync_copy(dst_hbm.at[sid, cn], dst_v.at[cpn])

                pltpu.async_copy(xl_hbm.at[cid].at[src_v.at[cpn, jn]],
                                 xj_v.at[1 - par], sgx[1 - par])
                pltpu.async_copy(xrm_hbm.at[cid].at[dst_v.at[cpn, jn]],
                                 xim_v.at[1 - par], sgm[1 - par])

                @pl.when(b >= 2)
                def _():
                    pltpu.make_async_copy(
                        ob_v.at[par], table.at[dst_v.at[cp, jr]],
                        ssc[par]).wait()

                compute(par, xj_v.at[par], xim_v.at[par], ob_v.at[par])

                pltpu.async_copy(ob_v.at[par], table.at[dst_v.at[cp, jr]],
                                 ssc[par], add=True)

        lb = NB - 2
        for par in range(2):
            b = lb + par
            cp = (b // CHUNK) % 2
            jr = b % CHUNK
            pltpu.make_async_copy(
                ob_v.at[par], table.at[dst_v.at[cp, jr]], ssc[par]).wait()
        cpo = (NB // CHUNK) % 2
        pltpu.make_async_copy(
            xl_hbm.at[cid].at[src_v.at[cpo, 0]], xj_v.at[0], sgx[0]).wait()
        pltpu.make_async_copy(
            xrm_hbm.at[cid].at[dst_v.at[cpo, 0]], xim_v.at[0], sgm[0]).wait()

        plsc.subcore_barrier()

        pltpu.sync_copy(
            table.at[pl.ds(sid * rows_per_tile, rows_per_tile)],
            out_hbm.at[cid, pl.ds(sid * rows_per_tile, rows_per_tile)])

    return k


def _make_gat_edge_sc2():
    C = D_OUT_K
    WR = C + 16
    WO = C + 8
    KD = 32
    NB2 = 648
    NBC = NB2 // 2
    CH2 = 12
    mesh = plsc.VectorSubcoreMesh(core_axis_name="c", subcore_axis_name="s")
    rows_per_tile = NPAD // NS

    @functools.partial(
        pl.kernel,
        out_type=jax.ShapeDtypeStruct((NC, NPAD, WO), jnp.float32),
        mesh=mesh,
        compiler_params=pltpu.CompilerParams(
            use_tc_tiling_on_sc=False, needs_layout_passes=False),
        scratch_types=[
            pltpu.VMEM((2, CH2, KD), jnp.int32),
            pltpu.VMEM((2, CH2, KD), jnp.int32),
            pltpu.VMEM((2, KD, C), jnp.float32),
            pltpu.VMEM((2, KD, WR), jnp.float32),
            pltpu.VMEM((2, KD, WO), jnp.float32),
            pltpu.VMEM((16, WO), jnp.float32),
            pltpu.VMEM((1, C + 16), jnp.float32),
            pltpu.VMEM_SHARED((NPAD, WO), jnp.float32),
            pltpu.SemaphoreType.DMA,
            pltpu.SemaphoreType.DMA,
            pltpu.SemaphoreType.DMA,
            pltpu.SemaphoreType.DMA,
            pltpu.SemaphoreType.DMA,
            pltpu.SemaphoreType.DMA,
        ],
    )
    def k(xl_hbm, xrm_hbm, src_hbm, dst_hbm, att_hbm, out_hbm,
          src_v, dst_v, xj_v, xim_v, ob_v, zb_v, att_v, table,
          sgx0, sgx1, sgm0, sgm1, ssc0, ssc1):
        cid = lax.axis_index("c")
        sid = lax.axis_index("s")
        sgx = (sgx0, sgx1)
        sgm = (sgm0, sgm1)
        ssc = (ssc0, ssc1)

        _zero_ref(zb_v, 16, WO)
        _zero_ref(ob_v.at[0], KD, WO)
        _zero_ref(ob_v.at[1], KD, WO)
        for t in range(rows_per_tile // 16):
            pltpu.sync_copy(zb_v, table.at[pl.ds(sid * rows_per_tile + t * 16, 16)])

        pltpu.sync_copy(att_hbm, att_v)

        plsc.subcore_barrier()

        start = cid * NBC
        c0 = start // CH2
        cp0 = c0 % 2
        pltpu.sync_copy(src_hbm.at[sid, c0], src_v.at[cp0])
        pltpu.sync_copy(dst_hbm.at[sid, c0], dst_v.at[cp0])
        pltpu.async_copy(xl_hbm.at[src_v.at[cp0, 0]], xj_v.at[0], sgx[0])
        pltpu.async_copy(xrm_hbm.at[dst_v.at[cp0, 0]], xim_v.at[0], sgm[0])

        def compute(xjr, ximr, obr):
            @pl.loop(0, KD // L)
            def _sub(sb):
                lanes = lax.iota(jnp.int32, L) + sb * L
                zeros_i = jnp.zeros((L,), jnp.int32)
                acc = jnp.zeros((L,), jnp.float32)
                for c in range(C):
                    col = jnp.full((L,), c, jnp.int32)
                    xjc = plsc.load_gather(xjr, [lanes, col])
                    xic = plsc.load_gather(ximr, [lanes, col])
                    attc = plsc.load_gather(
                        att_v, [zeros_i, jnp.full((L,), c + 16, jnp.int32)])
                    z = xjc + xic
                    lf = jnp.maximum(z, 0.2 * z)
                    acc = acc + attc * lf
                mh = plsc.load_gather(
                    ximr, [lanes, jnp.full((L,), C, jnp.int32)])
                ea = jnp.exp(acc - mh)
                plsc.store_scatter(
                    obr, [lanes, jnp.full((L,), C, jnp.int32)], ea)
                for c in range(C):
                    col = jnp.full((L,), c, jnp.int32)
                    xjc = plsc.load_gather(xjr, [lanes, col])
                    plsc.store_scatter(obr, [lanes, col], xjc * ea)

        @pl.loop(start, start + NBC, step=2)
        def _pair(i):
            for par in range(2):
                b = i + par
                cp = (b // CH2) % 2
                jr = b % CH2
                bn = b + 1
                cn = bn // CH2
                jn = bn % CH2
                cpn = cn % 2
                pltpu.make_async_copy(
                    xl_hbm.at[src_v.at[cp, jr]], xj_v.at[par], sgx[par]).wait()
                pltpu.make_async_copy(
                    xrm_hbm.at[dst_v.at[cp, jr]], xim_v.at[par], sgm[par]).wait()

                @pl.when(jn == 0)
                def _():
                    pltpu.sync_copy(src_hbm.at[sid, cn], src_v.at[cpn])
                    pltpu.sync_copy(dst_hbm.at[sid, cn], dst_v.at[cpn])

                pltpu.async_copy(xl_hbm.at[src_v.at[cpn, jn]],
                                 xj_v.at[1 - par], sgx[1 - par])
                pltpu.async_copy(xrm_hbm.at[dst_v.at[cpn, jn]],
                                 xim_v.at[1 - par], sgm[1 - par])

                @pl.when(b >= start + 2)
                def _():
                    pltpu.make_async_copy(
                        ob_v.at[par], table.at[dst_v.at[cp, jr]],
                        ssc[par]).wait()

                compute(xj_v.at[par], xim_v.at[par], ob_v.at[par])

                pltpu.async_copy(ob_v.at[par], table.at[dst_v.at[cp, jr]],
                                 ssc[par], add=True)

        for par in range(2):
            pltpu.make_async_copy(
                ob_v.at[par], table.at[dst_v.at[0, 0]], ssc[par]).wait()
        pltpu.make_async_copy(
            xl_hbm.at[src_v.at[0, 0]], xj_v.at[0], sgx[0]).wait()
        pltpu.make_async_copy(
            xrm_hbm.at[dst_v.at[0, 0]], xim_v.at[0], sgm[0]).wait()

        plsc.subcore_barrier()

        pltpu.sync_copy(
            table.at[pl.ds(sid * rows_per_tile, rows_per_tile)],
            out_hbm.at[cid, pl.ds(sid * rows_per_tile, rows_per_tile)])

    return k


def _make_epilogue01(blk=1024):
    H, C = HEADS_K, HID_K
    HC = H * C
    HW = HC // 2
    WO = HW + 8
    grid = NPAD // blk

    def body(parts_ref, res_ref, bo_ref, g_ref, be_ref, h_ref):
        p0 = parts_ref[0]
        p1 = parts_ref[1]
        acc = jnp.concatenate([p0[:, :HW], p1[:, :HW]], axis=1)
        den = jnp.concatenate([p0[:, HW:HW + 3], p1[:, HW:HW + 3]], axis=1)
        o = acc.reshape(blk, H, C) / (den.reshape(blk, H, 1) + 1e-30)
        o = o.reshape(blk, HC) + bo_ref[...]
        t = o + res_ref[...]
        mu = jnp.mean(t, axis=-1, keepdims=True)
        var = jnp.mean((t - mu) ** 2, axis=-1, keepdims=True)
        y = (t - mu) / jnp.sqrt(var + 1e-5) * g_ref[...] + be_ref[...]
        h_ref[...] = jnp.where(y > 0, y, jnp.exp(y) - 1.0)

    return pl.pallas_call(
        body,
        grid=(grid,),
        in_specs=[
            pl.BlockSpec((NC, blk, WO), lambda i: (0, i, 0)),
            pl.BlockSpec((blk, HC), lambda i: (i, 0)),
            pl.BlockSpec((1, HC), lambda i: (0, 0)),
            pl.BlockSpec((1, HC), lambda i: (0, 0)),
            pl.BlockSpec((1, HC), lambda i: (0, 0)),
        ],
        out_specs=pl.BlockSpec((blk, HC), lambda i: (i, 0)),
        out_shape=jax.ShapeDtypeStruct((NPAD, HC), jnp.float32),
    )


def _make_final_epilogue(blk=1024):
    C = D_OUT_K
    WO = C + 8
    grid = NPAD // blk

    def body(parts_ref, bo_ref, out_ref):
        p = parts_ref[0] + parts_ref[1]
        out_ref[...] = p[:, :C] / (p[:, C:C + 1] + 1e-30) + bo_ref[...]

    return pl.pallas_call(
        body,
        grid=(grid,),
        in_specs=[
            pl.BlockSpec((NC, blk, WO), lambda i: (0, i, 0)),
            pl.BlockSpec((1, C), lambda i: (0, 0)),
        ],
        out_specs=pl.BlockSpec((blk, C), lambda i: (i, 0)),
        out_shape=jax.ShapeDtypeStruct((NPAD, C), jnp.float32),
    )


_mm0 = _make_matmul_pack01(D_IN_K)
_mm1 = _make_matmul_pack01(HC_K)
_mm2 = _make_matmul_pack2(HC_K)
_sc01 = _make_gat_edge_sc01()
_sc2 = _make_gat_edge_sc2()
_epi01 = _make_epilogue01()
_epi2 = _make_final_epilogue()


def kernel(x, edge_index, Wl0, bl0, Wr0, br0, att0, bo0, rW0, rb0, g0, be0,
           Wl1, bl1, Wr1, br1, att1, bo1, rW1, rb1, g1, be1,
           Wl2, bl2, Wr2, br2, att2, bo2):
    n = x.shape[0]
    si = jnp.arange(n, dtype=jnp.int32)
    pad_ids = jnp.full((EPAD - E_SELF,), n, jnp.int32)

    def _edge_arr(e, kd, nch):
        per_tile = jnp.concatenate([e, si, pad_ids]).reshape(NS, T_EDGES)
        dummy = jnp.full((NS, CHUNK * kd), n, jnp.int32)
        return jnp.concatenate([per_tile, dummy], axis=1).reshape(
            NS, nch + 1, CHUNK, kd)

    src2 = _edge_arr(edge_index[0], KDMA, NCHUNK)
    dst2 = _edge_arr(edge_index[1], KDMA, NCHUNK)
    src2b = _edge_arr(edge_index[0], 32, 54)
    dst2b = _edge_arr(edge_index[1], 32, 54)

    xp = jnp.pad(x, ((0, NPAD - n), (0, 0)))

    xl, xrm, res = _mm0(xp, Wl0, bl0.reshape(1, -1), Wr0, br0.reshape(1, -1),
                        att0, rW0, rb0.reshape(1, -1))
    parts = _sc01(xl, xrm, src2, dst2, att0)
    h = _epi01(parts, res, bo0.reshape(1, -1), g0.reshape(1, -1),
               be0.reshape(1, -1))

    xl, xrm, res = _mm1(h, Wl1, bl1.reshape(1, -1), Wr1, br1.reshape(1, -1),
                        att1, rW1, rb1.reshape(1, -1))
    parts = _sc01(xl, xrm, src2, dst2, att1)
    h = _epi01(parts, res, bo1.reshape(1, -1), g1.reshape(1, -1),
               be1.reshape(1, -1))

    xl, xrm = _mm2(h, Wl2, bl2.reshape(1, -1), Wr2, br2.reshape(1, -1), att2)
    att2p = jnp.pad(att2, ((0, 0), (16, 0)))
    parts = _sc2(xl, xrm, src2b, dst2b, att2p)
    out = _epi2(parts, bo2.reshape(1, -1))
    return out[:n]
